# initial kernel scaffold (unmeasured)
import jax
import jax.numpy as jnp
from jax import lax
from jax.experimental import pallas as pl
from jax.experimental.pallas import tpu as pltpu

N_DEV = 4
M_PER = 1024
K_PER = 1024
N_OUT = 8192
NT = 512
NE = 1024


def kernel(x, w_mat):
    assert x.shape == (N_DEV * M_PER, K_PER), x.shape
    assert w_mat.shape == (N_DEV * K_PER, N_OUT), w_mat.shape

    def body(x_hbm, w_hbm, out_ref, send_buf, recv_buf, stage, w_buf,
             amax_buf, send_sems, recv_sems, a_send_sems, a_recv_sems,
             w_sems, x_sem):
        my = lax.axis_index("i")

        barrier = pltpu.get_barrier_semaphore()
        for h in (1, 2, 3):
            pl.semaphore_signal(
                barrier, inc=1,
                device_id=((my + h) % N_DEV,),
                device_id_type=pl.DeviceIdType.MESH,
            )
        pl.semaphore_wait(barrier, N_DEV - 1)

        data_rdmas = []
        for o, h in enumerate((1, 3, 2)):
            p = (my + h) % N_DEV
            cp = pltpu.make_async_copy(
                x_hbm.at[pl.ds(p * M_PER, M_PER), :], stage, x_sem)
            cp.start()
            cp.wait()
            send_buf[o] = stage[...].astype(jnp.bfloat16)
            rdma = pltpu.make_async_remote_copy(
                src_ref=send_buf.at[o],
                dst_ref=recv_buf.at[3 - h],
                send_sem=send_sems.at[o],
                recv_sem=recv_sems.at[3 - h],
                device_id=(p,),
                device_id_type=pl.DeviceIdType.MESH,
            )
            rdma.start()
            data_rdmas.append(rdma)
        cp = pltpu.make_async_copy(
            x_hbm.at[pl.ds(my * M_PER, M_PER), :], stage, x_sem)
        cp.start()
        cp.wait()
        send_buf[3] = stage[...].astype(jnp.bfloat16)

        passes = [
            (send_buf, 3, my),
            (recv_buf, 0, (my + 1) % N_DEV),
            (recv_buf, 2, (my + 3) % N_DEV),
            (recv_buf, 1, (my + 2) % N_DEV),
        ]
        n_tiles = N_OUT // NT
        total = len(passes) * n_tiles

        def start_w_copy(t):
            if t < total:
                j = passes[t // n_tiles][2]
                nt = t % n_tiles
                c = pltpu.make_async_copy(
                    w_hbm.at[pl.ds(j * K_PER, K_PER), pl.ds(nt * NT, NT)],
                    w_buf.at[t % 3],
                    w_sems.at[t % 3],
                )
                c.start()
                return c
            return None

        copies = {0: start_w_copy(0), 1: start_w_copy(1)}
        for t in range(total):
            pi, nt = t // n_tiles, t % n_tiles
            a_ref, slot, _ = passes[pi]
            if nt == 0 and pi > 0:
                pltpu.make_async_remote_copy(
                    src_ref=recv_buf.at[slot], dst_ref=recv_buf.at[slot],
                    send_sem=send_sems.at[0], recv_sem=recv_sems.at[slot],
                    device_id=(my,), device_id_type=pl.DeviceIdType.MESH,
                ).wait_recv()
            copies[t].wait()
            copies[t + 2] = start_w_copy(t + 2)
            wt = w_buf[t % 3].astype(jnp.bfloat16)
            acc = jnp.dot(a_ref[slot], wt, preferred_element_type=jnp.float32)
            ns = pl.ds(nt * NT, NT)
            if pi == 0:
                out_ref[:, ns] = acc
            else:
                out_ref[:, ns] = out_ref[:, ns] + acc

        m = jnp.float32(0)
        for e in range(N_OUT // NE):
            m = jnp.maximum(m, jnp.max(jnp.abs(out_ref[:, pl.ds(e * NE, NE)])))
        amax_buf[pl.ds(my, 1)] = jnp.broadcast_to(m, (1, 8, 128))
        amax_rdmas = []
        for h in (1, 3, 2):
            p = (my + h) % N_DEV
            r = pltpu.make_async_remote_copy(
                src_ref=amax_buf.at[pl.ds(my, 1)],
                dst_ref=amax_buf.at[pl.ds(my, 1)],
                send_sem=a_send_sems.at[3 - h],
                recv_sem=a_recv_sems.at[3 - h],
                device_id=(p,),
                device_id_type=pl.DeviceIdType.MESH,
            )
            r.start()
            amax_rdmas.append(r)
        for s in range(3):
            j = (my + s + 1) % N_DEV
            pltpu.make_async_remote_copy(
                src_ref=amax_buf.at[pl.ds(j, 1)],
                dst_ref=amax_buf.at[pl.ds(j, 1)],
                send_sem=a_send_sems.at[s],
                recv_sem=a_recv_sems.at[s],
                device_id=(my,), device_id_type=pl.DeviceIdType.MESH,
            ).wait_recv()
        g = jnp.max(amax_buf[...])

        scale = g * (1.0 / 448.0)
        inv = 448.0 / g
        for e in range(N_OUT // NE):
            ns = pl.ds(e * NE, NE)
            v = jnp.clip(out_ref[:, ns] * inv, -448.0, 448.0)
            q = v.astype(jnp.float8_e4m3fn).astype(jnp.float32)
            out_ref[:, ns] = q * scale

        for r in data_rdmas + amax_rdmas:
            r.wait_send()

    return pl.pallas_call(
        body,
        out_shape=jax.ShapeDtypeStruct((M_PER, N_OUT), jnp.float32),
        in_specs=[
            pl.BlockSpec(memory_space=pltpu.ANY),
            pl.BlockSpec(memory_space=pltpu.ANY),
        ],
        out_specs=pl.BlockSpec(memory_space=pltpu.VMEM),
        scratch_shapes=[
            pltpu.VMEM((4, M_PER, K_PER), jnp.bfloat16),
            pltpu.VMEM((3, M_PER, K_PER), jnp.bfloat16),
            pltpu.VMEM((M_PER, K_PER), jnp.float32),
            pltpu.VMEM((3, K_PER, NT), jnp.float32),
            pltpu.VMEM((N_DEV, 8, 128), jnp.float32),
            pltpu.SemaphoreType.DMA((3,)),
            pltpu.SemaphoreType.DMA((3,)),
            pltpu.SemaphoreType.DMA((3,)),
            pltpu.SemaphoreType.DMA((3,)),
            pltpu.SemaphoreType.DMA((3,)),
            pltpu.SemaphoreType.DMA,
        ],
        compiler_params=pltpu.CompilerParams(collective_id=0),
    )(x, w_mat)


# baseline (device time: 146598 ns/iter reference)
import jax
import jax.numpy as jnp
from jax import lax
from jax.experimental import pallas as pl
from jax.experimental.pallas import tpu as pltpu

N_DEV = 4
M_PER = 1024
K_PER = 1024
N_OUT = 8192
NT = 512
NE = 1024


def kernel(x, w_mat):
    assert x.shape == (N_DEV * M_PER, K_PER), x.shape
    assert w_mat.shape == (N_DEV * K_PER, N_OUT), w_mat.shape

    def body(x_hbm, w_hbm, out_ref, send_buf, recv_buf, stage, w_buf,
             amax_buf, send_sems, recv_sems, a_send_sems, a_recv_sems,
             w_sems, x_sem):
        my = lax.axis_index("i")

        barrier = pltpu.get_barrier_semaphore()
        for h in (1, 2, 3):
            pl.semaphore_signal(
                barrier, inc=1,
                device_id=((my + h) % N_DEV,),
                device_id_type=pl.DeviceIdType.MESH,
            )
        pl.semaphore_wait(barrier, N_DEV - 1)

        data_rdmas = []
        for o, h in enumerate((1, 3, 2)):
            p = (my + h) % N_DEV
            cp = pltpu.make_async_copy(
                x_hbm.at[pl.ds(p * M_PER, M_PER), :], stage, x_sem)
            cp.start()
            cp.wait()
            send_buf[o] = stage[...].astype(jnp.bfloat16)
            rdma = pltpu.make_async_remote_copy(
                src_ref=send_buf.at[o],
                dst_ref=recv_buf.at[3 - h],
                send_sem=send_sems.at[o],
                recv_sem=recv_sems.at[3 - h],
                device_id=(p,),
                device_id_type=pl.DeviceIdType.MESH,
            )
            rdma.start()
            data_rdmas.append(rdma)
        cp = pltpu.make_async_copy(
            x_hbm.at[pl.ds(my * M_PER, M_PER), :], stage, x_sem)
        cp.start()
        cp.wait()
        send_buf[3] = stage[...].astype(jnp.bfloat16)

        passes = [
            (send_buf, 3, my),
            (recv_buf, 0, (my + 1) % N_DEV),
            (recv_buf, 2, (my + 3) % N_DEV),
            (recv_buf, 1, (my + 2) % N_DEV),
        ]
        n_tiles = N_OUT // NT
        total = len(passes) * n_tiles

        def start_w_copy(t):
            if t < total:
                j = passes[t // n_tiles][2]
                nt = t % n_tiles
                c = pltpu.make_async_copy(
                    w_hbm.at[pl.ds(j * K_PER, K_PER), pl.ds(nt * NT, NT)],
                    w_buf.at[t % 3],
                    w_sems.at[t % 3],
                )
                c.start()
                return c
            return None

        copies = {0: start_w_copy(0), 1: start_w_copy(1)}
        for t in range(total):
            pi, nt = t // n_tiles, t % n_tiles
            a_ref, slot, _ = passes[pi]
            if nt == 0 and pi > 0:
                pltpu.make_async_remote_copy(
                    src_ref=recv_buf.at[slot], dst_ref=recv_buf.at[slot],
                    send_sem=send_sems.at[0], recv_sem=recv_sems.at[slot],
                    device_id=(my,), device_id_type=pl.DeviceIdType.MESH,
                ).wait_recv()
            copies[t].wait()
            copies[t + 2] = start_w_copy(t + 2)
            wt = w_buf[t % 3].astype(jnp.bfloat16)
            acc = jnp.dot(a_ref[slot], wt, preferred_element_type=jnp.float32)
            ns = pl.ds(nt * NT, NT)
            if pi == 0:
                out_ref[:, ns] = acc
            else:
                out_ref[:, ns] = out_ref[:, ns] + acc

        m = jnp.float32(0)
        for e in range(N_OUT // NE):
            m = jnp.maximum(m, jnp.max(jnp.abs(out_ref[:, pl.ds(e * NE, NE)])))
        amax_buf[pl.ds(my, 1)] = jnp.broadcast_to(m, (1, 8, 128))
        amax_rdmas = []
        for h in (1, 3, 2):
            p = (my + h) % N_DEV
            r = pltpu.make_async_remote_copy(
                src_ref=amax_buf.at[pl.ds(my, 1)],
                dst_ref=amax_buf.at[pl.ds(my, 1)],
                send_sem=a_send_sems.at[3 - h],
                recv_sem=a_recv_sems.at[3 - h],
                device_id=(p,),
                device_id_type=pl.DeviceIdType.MESH,
            )
            r.start()
            amax_rdmas.append(r)
        for s in range(3):
            j = (my + s + 1) % N_DEV
            pltpu.make_async_remote_copy(
                src_ref=amax_buf.at[pl.ds(j, 1)],
                dst_ref=amax_buf.at[pl.ds(j, 1)],
                send_sem=a_send_sems.at[s],
                recv_sem=a_recv_sems.at[s],
                device_id=(my,), device_id_type=pl.DeviceIdType.MESH,
            ).wait_recv()
        g = jnp.max(amax_buf[...])

        scale = g * (1.0 / 448.0)
        inv = 448.0 / g
        for e in range(N_OUT // NE):
            ns = pl.ds(e * NE, NE)
            v = jnp.clip(out_ref[:, ns] * inv, -448.0, 448.0)
            q = v.astype(jnp.float8_e4m3fn).astype(jnp.float32)
            out_ref[:, ns] = q * scale

        for r in data_rdmas + amax_rdmas:
            r.wait_send()

    return pl.pallas_call(
        body,
        out_shape=jax.ShapeDtypeStruct((M_PER, N_OUT), jnp.float32),
        in_specs=[
            pl.BlockSpec(memory_space=pl.ANY),
            pl.BlockSpec(memory_space=pl.ANY),
        ],
        out_specs=pl.BlockSpec(memory_space=pltpu.VMEM),
        scratch_shapes=[
            pltpu.VMEM((4, M_PER, K_PER), jnp.bfloat16),
            pltpu.VMEM((3, M_PER, K_PER), jnp.bfloat16),
            pltpu.VMEM((M_PER, K_PER), jnp.float32),
            pltpu.VMEM((3, K_PER, NT), jnp.float32),
            pltpu.VMEM((N_DEV, 8, 128), jnp.float32),
            pltpu.SemaphoreType.DMA((3,)),
            pltpu.SemaphoreType.DMA((3,)),
            pltpu.SemaphoreType.DMA((3,)),
            pltpu.SemaphoreType.DMA((3,)),
            pltpu.SemaphoreType.DMA((3,)),
            pltpu.SemaphoreType.DMA,
        ],
        compiler_params=pltpu.CompilerParams(
            collective_id=0,
            vmem_limit_bytes=64 * 1024 * 1024,
        ),
    )(x, w_mat)
